# baseline (device time: 42456 ns/iter reference)
import jax
import jax.numpy as jnp
from jax import lax
from jax.experimental import pallas as pl
from jax.experimental.pallas import tpu as pltpu

N_DEV = 8
B, SQ, D = 4, 256, 1024
ROWS = B * SQ
HQ_LOC, HKV_LOC, DH = 8, 2, 128
SCALE = 0.08838834764831843

G = 4
GROWS = ROWS // G
OWN = GROWS // N_DEV

PART_COLS = ((0, 384), (384, 384), (768, 256))
ORDER = ((0, 1, 2), (1, 2, 0), (2, 0, 1))
RS_SLOT = (0, 128, 192)
GRECV = 224


def kernel(x, Wq, Wo, Wk, Wv):
    idx = lax.axis_index("i")
    wk_s = lax.dynamic_slice_in_dim(Wk, idx * (HKV_LOC * DH), HKV_LOC * DH, axis=1)
    wv_s = lax.dynamic_slice_in_dim(Wv, idx * (HKV_LOC * DH), HKV_LOC * DH, axis=1)

    def body(x_ref, wq_ref, wo_ref, wk_ref, wv_ref, out_ref,
             attn_ref, q_ref, k_ref, v_ref,
             p0, p1, p2, r0, r1, r2, g0, g1, g2,
             s0, s1, s2, send_sems, recv_sems):
        p_refs = (p0, p1, p2)
        r_refs = (r0, r1, r2)
        g_refs = (g0, g1, g2)
        s_refs = (s0, s1, s2)

        i = lax.axis_index("i")
        partners = (
            jnp.bitwise_xor(i, 1),
            jnp.bitwise_xor(i, 3),
            jnp.bitwise_xor(i, 4),
        )
        lows = (
            jnp.bitwise_and(jnp.bitwise_xor(i, i >> 1), 1) == 0,
            jnp.bitwise_and(i >> 1, 1) == 0,
            jnp.bitwise_and(i >> 2, 1) == 0,
        )

        barrier_sem = pltpu.get_barrier_semaphore()
        for pd in partners:
            pl.semaphore_signal(barrier_sem, inc=1, device_id=(pd,),
                                device_id_type=pl.DeviceIdType.MESH)
        pl.semaphore_wait(barrier_sem, 3)

        def rs_sem(g, k, p):
            return g * 18 + k * 3 + p

        def ag_sem(g, k, p):
            return g * 18 + 9 + k * 3 + p

        def make_rs(g, k, p, src_off, half):
            s_refs[p][pl.ds(g * GRECV + RS_SLOT[k], half), :] = (
                p_refs[p][pl.ds(pl.multiple_of(src_off, OWN), half), :]
                .astype(jnp.bfloat16))
            return pltpu.make_async_remote_copy(
                src_ref=s_refs[p].at[pl.ds(g * GRECV + RS_SLOT[k], half), :],
                dst_ref=r_refs[p].at[pl.ds(g * GRECV + RS_SLOT[k], half), :],
                send_sem=send_sems.at[rs_sem(g, k, p)],
                recv_sem=recv_sems.at[rs_sem(g, k, p)],
                device_id=(partners[ORDER[p][k]],),
                device_id_type=pl.DeviceIdType.MESH,
            )

        def make_ag(g, k, p, off, size):
            return pltpu.make_async_remote_copy(
                src_ref=g_refs[p].at[pl.ds(pl.multiple_of(off, OWN), size), :],
                dst_ref=g_refs[p].at[pl.ds(pl.multiple_of(off, OWN), size), :],
                send_sem=send_sems.at[ag_sem(g, k, p)],
                recv_sem=recv_sems.at[ag_sem(g, k, p)],
                device_id=(partners[ORDER[p][2 - k]],),
                device_id_type=pl.DeviceIdType.MESH,
            )

        offs = [[jnp.int32(g * GROWS)] * 3 for g in range(G)]
        rs_pend = [[None] * 3 for _ in range(G)]
        ag_pend = [[None] * 3 for _ in range(G)]

        xf = x_ref[...].reshape(ROWS, D).astype(jnp.bfloat16)
        wq16 = wq_ref[:].astype(jnp.bfloat16)
        wk16 = wk_ref[:].astype(jnp.bfloat16)
        wv16 = wv_ref[:].astype(jnp.bfloat16)
        q_ref[:] = jnp.dot(xf, wq16,
                           preferred_element_type=jnp.float32
                           ).astype(jnp.bfloat16)
        k_ref[:] = jnp.dot(xf, wk16,
                           preferred_element_type=jnp.float32
                           ).astype(jnp.bfloat16)
        v_ref[:] = jnp.dot(xf, wv16,
                           preferred_element_type=jnp.float32
                           ).astype(jnp.bfloat16)
        wo16 = wo_ref[:].astype(jnp.bfloat16)

        for g in range(G):
            b = g
            qb = q_ref[b * SQ:(b + 1) * SQ, :]
            kb = k_ref[b * SQ:(b + 1) * SQ, :]
            vb = v_ref[b * SQ:(b + 1) * SQ, :]
            for h in range(HQ_LOC):
                kv = h // 4
                q = qb[:, h * DH:(h + 1) * DH]
                k_ = kb[:, kv * DH:(kv + 1) * DH]
                v_ = vb[:, kv * DH:(kv + 1) * DH]
                s = lax.dot_general(
                    q, k_, (((1,), (1,)), ((), ())),
                    preferred_element_type=jnp.float32,
                ) * SCALE
                m = jnp.max(s, axis=1, keepdims=True)
                pe = jnp.exp(s - m)
                l = jnp.sum(pe, axis=1, keepdims=True)
                o = jnp.dot(pe.astype(jnp.bfloat16), v_,
                            preferred_element_type=jnp.float32) / l
                attn_ref[b * SQ:(b + 1) * SQ, h * DH:(h + 1) * DH] = (
                    o.astype(jnp.bfloat16))
            rb = g * GROWS
            half = GROWS // 2
            for p in range(3):
                c0, nc = PART_COLS[p]
                p_refs[p][rb:rb + GROWS, :] = jnp.dot(
                    attn_ref[rb:rb + GROWS, :], wo16[:, c0:c0 + nc],
                    preferred_element_type=jnp.float32)
                low = lows[ORDER[p][0]]
                send_off = jnp.where(low, rb + half, rb)
                rdma = make_rs(g, 0, p, send_off, half)
                rdma.start()
                rs_pend[g][p] = (rdma, jnp.where(low, rb, rb + half))

        for t in range(6):
            for g in range(G):
                if t < 3:
                    k = t
                    half = GROWS >> (k + 1)
                    for p in range(3):
                        rdma, keep_off = rs_pend[g][p]
                        rdma.wait()
                        p_refs[p][pl.ds(pl.multiple_of(keep_off, OWN), half), :] = (
                            p_refs[p][pl.ds(pl.multiple_of(keep_off, OWN), half), :]
                            + r_refs[p][pl.ds(g * GRECV + RS_SLOT[k], half), :]
                            .astype(jnp.float32)
                        )
                        offs[g][p] = keep_off
                    if k < 2:
                        nhalf = half // 2
                        for p in range(3):
                            low = lows[ORDER[p][k + 1]]
                            send_off = jnp.where(low, offs[g][p] + nhalf,
                                                 offs[g][p])
                            rdma = make_rs(g, k + 1, p, send_off, nhalf)
                            rdma.start()
                            rs_pend[g][p] = (rdma,
                                             jnp.where(low, offs[g][p],
                                                       offs[g][p] + nhalf))
                    else:
                        for p in range(3):
                            g_refs[p][pl.ds(pl.multiple_of(offs[g][p], OWN), OWN), :] = (
                                p_refs[p][pl.ds(pl.multiple_of(offs[g][p], OWN), OWN), :]
                                .astype(jnp.bfloat16))
                            rdma = make_ag(g, 0, p, offs[g][p], OWN)
                            rdma.start()
                            ag_pend[g][p] = rdma
                else:
                    k = t - 3
                    size = OWN << k
                    for p in range(3):
                        ag_pend[g][p].wait()
                        low = lows[ORDER[p][2 - k]]
                        offs[g][p] = jnp.where(low, offs[g][p],
                                               offs[g][p] - size)
                    if k < 2:
                        for p in range(3):
                            rdma = make_ag(g, k + 1, p, offs[g][p], size * 2)
                            rdma.start()
                            ag_pend[g][p] = rdma
                    else:
                        rb = g * GROWS
                        for p in range(3):
                            c0, nc = PART_COLS[p]
                            out_ref[rb:rb + GROWS, c0:c0 + nc] = (
                                g_refs[p][rb:rb + GROWS, :]
                                .astype(jnp.float32))

    flat = pl.pallas_call(
        body,
        out_shape=jax.ShapeDtypeStruct((ROWS, D), jnp.float32),
        in_specs=[pl.BlockSpec(memory_space=pltpu.VMEM)] * 5,
        out_specs=pl.BlockSpec(memory_space=pltpu.VMEM),
        scratch_shapes=[
            pltpu.VMEM((ROWS, D), jnp.bfloat16),
            pltpu.VMEM((ROWS, D), jnp.bfloat16),
            pltpu.VMEM((ROWS, HKV_LOC * DH), jnp.bfloat16),
            pltpu.VMEM((ROWS, HKV_LOC * DH), jnp.bfloat16),
            pltpu.VMEM((ROWS, 384), jnp.float32),
            pltpu.VMEM((ROWS, 384), jnp.float32),
            pltpu.VMEM((ROWS, 256), jnp.float32),
            pltpu.VMEM((G * GRECV, 384), jnp.bfloat16),
            pltpu.VMEM((G * GRECV, 384), jnp.bfloat16),
            pltpu.VMEM((G * GRECV, 256), jnp.bfloat16),
            pltpu.VMEM((ROWS, 384), jnp.bfloat16),
            pltpu.VMEM((ROWS, 384), jnp.bfloat16),
            pltpu.VMEM((ROWS, 256), jnp.bfloat16),
            pltpu.VMEM((G * GRECV, 384), jnp.bfloat16),
            pltpu.VMEM((G * GRECV, 384), jnp.bfloat16),
            pltpu.VMEM((G * GRECV, 256), jnp.bfloat16),
            pltpu.SemaphoreType.DMA((G * 18,)),
            pltpu.SemaphoreType.DMA((G * 18,)),
        ],
        compiler_params=pltpu.CompilerParams(collective_id=0),
    )(x, Wq, Wo, wk_s, wv_s)
    return flat.reshape(B, SQ, D)


# device time: 41986 ns/iter; 1.0112x vs baseline; 1.0112x over previous
import jax
import jax.numpy as jnp
from jax import lax
from jax.experimental import pallas as pl
from jax.experimental.pallas import tpu as pltpu

N_DEV = 8
B, SQ, D = 4, 256, 1024
ROWS = B * SQ
HQ_LOC, HKV_LOC, DH = 8, 2, 128
SCALE = 0.08838834764831843

G = 4
GROWS = ROWS // G
OWN = GROWS // N_DEV

PART_COLS = ((0, 384), (384, 384), (768, 256))
ORDER = ((0, 1, 2), (1, 2, 0), (2, 0, 1))
RS_SLOT = (0, 128, 192)
GRECV = 224


def kernel(x, Wq, Wo, Wk, Wv):
    idx = lax.axis_index("i")
    wk_s = lax.dynamic_slice_in_dim(Wk, idx * (HKV_LOC * DH), HKV_LOC * DH, axis=1)
    wv_s = lax.dynamic_slice_in_dim(Wv, idx * (HKV_LOC * DH), HKV_LOC * DH, axis=1)

    def body(x_ref, wq_ref, wo_ref, wk_ref, wv_ref, out_ref,
             attn_ref, q_ref, k_ref, v_ref,
             p0, p1, p2, r0, r1, r2, g0, g1, g2,
             s0, s1, s2, send_sems, recv_sems):
        p_refs = (p0, p1, p2)
        r_refs = (r0, r1, r2)
        g_refs = (g0, g1, g2)
        s_refs = (s0, s1, s2)

        i = lax.axis_index("i")
        partners = (
            jnp.bitwise_xor(i, 1),
            jnp.bitwise_xor(i, 3),
            jnp.bitwise_xor(i, 4),
        )
        lows = (
            jnp.bitwise_and(jnp.bitwise_xor(i, i >> 1), 1) == 0,
            jnp.bitwise_and(i >> 1, 1) == 0,
            jnp.bitwise_and(i >> 2, 1) == 0,
        )

        barrier_sem = pltpu.get_barrier_semaphore()
        for pd in partners:
            pl.semaphore_signal(barrier_sem, inc=1, device_id=(pd,),
                                device_id_type=pl.DeviceIdType.MESH)
        pl.semaphore_wait(barrier_sem, 3)

        def rs_sem(g, k, p):
            return g * 18 + k * 3 + p

        def ag_sem(g, k, p):
            return g * 18 + 9 + k * 3 + p

        def make_rs(g, k, p, src_off, half):
            s_refs[p][pl.ds(g * GRECV + RS_SLOT[k], half), :] = (
                p_refs[p][pl.ds(pl.multiple_of(src_off, OWN), half), :]
                .astype(jnp.bfloat16))
            return pltpu.make_async_remote_copy(
                src_ref=s_refs[p].at[pl.ds(g * GRECV + RS_SLOT[k], half), :],
                dst_ref=r_refs[p].at[pl.ds(g * GRECV + RS_SLOT[k], half), :],
                send_sem=send_sems.at[rs_sem(g, k, p)],
                recv_sem=recv_sems.at[rs_sem(g, k, p)],
                device_id=(partners[ORDER[p][k]],),
                device_id_type=pl.DeviceIdType.MESH,
            )

        def make_ag(g, k, p, off, size):
            return pltpu.make_async_remote_copy(
                src_ref=g_refs[p].at[pl.ds(pl.multiple_of(off, OWN), size), :],
                dst_ref=g_refs[p].at[pl.ds(pl.multiple_of(off, OWN), size), :],
                send_sem=send_sems.at[ag_sem(g, k, p)],
                recv_sem=recv_sems.at[ag_sem(g, k, p)],
                device_id=(partners[ORDER[p][2 - k]],),
                device_id_type=pl.DeviceIdType.MESH,
            )

        offs = [[jnp.int32(g * GROWS)] * 3 for g in range(G)]
        rs_pend = [[None] * 3 for _ in range(G)]
        ag_pend = [[None] * 3 for _ in range(G)]

        xf = x_ref[...].reshape(ROWS, D)
        q_ref[:] = jnp.dot(xf, wq_ref[:], preferred_element_type=jnp.float32)
        k_ref[:] = jnp.dot(xf, wk_ref[:], preferred_element_type=jnp.float32)
        v_ref[:] = jnp.dot(xf, wv_ref[:], preferred_element_type=jnp.float32)

        for g in range(G):
            b = g
            qb = q_ref[b * SQ:(b + 1) * SQ, :]
            kb = k_ref[b * SQ:(b + 1) * SQ, :]
            vb = v_ref[b * SQ:(b + 1) * SQ, :]
            for h in range(HQ_LOC):
                kv = h // 4
                q = qb[:, h * DH:(h + 1) * DH]
                k_ = kb[:, kv * DH:(kv + 1) * DH]
                v_ = vb[:, kv * DH:(kv + 1) * DH]
                s = lax.dot_general(
                    q, k_, (((1,), (1,)), ((), ())),
                    preferred_element_type=jnp.float32,
                ) * SCALE
                m = jnp.max(s, axis=1, keepdims=True)
                pe = jnp.exp(s - m)
                l = jnp.sum(pe, axis=1, keepdims=True)
                o = jnp.dot(pe, v_, preferred_element_type=jnp.float32) / l
                attn_ref[b * SQ:(b + 1) * SQ, h * DH:(h + 1) * DH] = o
            rb = g * GROWS
            half = GROWS // 2
            for p in range(3):
                c0, nc = PART_COLS[p]
                p_refs[p][rb:rb + GROWS, :] = jnp.dot(
                    attn_ref[rb:rb + GROWS, :], wo_ref[:, c0:c0 + nc],
                    preferred_element_type=jnp.float32)
                low = lows[ORDER[p][0]]
                send_off = jnp.where(low, rb + half, rb)
                rdma = make_rs(g, 0, p, send_off, half)
                rdma.start()
                rs_pend[g][p] = (rdma, jnp.where(low, rb, rb + half))

        for t in range(6):
            for g in range(G):
                if t < 3:
                    k = t
                    half = GROWS >> (k + 1)
                    for p in range(3):
                        rdma, keep_off = rs_pend[g][p]
                        rdma.wait()
                        p_refs[p][pl.ds(pl.multiple_of(keep_off, OWN), half), :] = (
                            p_refs[p][pl.ds(pl.multiple_of(keep_off, OWN), half), :]
                            + r_refs[p][pl.ds(g * GRECV + RS_SLOT[k], half), :]
                            .astype(jnp.float32)
                        )
                        offs[g][p] = keep_off
                    if k < 2:
                        nhalf = half // 2
                        for p in range(3):
                            low = lows[ORDER[p][k + 1]]
                            send_off = jnp.where(low, offs[g][p] + nhalf,
                                                 offs[g][p])
                            rdma = make_rs(g, k + 1, p, send_off, nhalf)
                            rdma.start()
                            rs_pend[g][p] = (rdma,
                                             jnp.where(low, offs[g][p],
                                                       offs[g][p] + nhalf))
                    else:
                        for p in range(3):
                            g_refs[p][pl.ds(pl.multiple_of(offs[g][p], OWN), OWN), :] = (
                                p_refs[p][pl.ds(pl.multiple_of(offs[g][p], OWN), OWN), :]
                                .astype(jnp.bfloat16))
                            rdma = make_ag(g, 0, p, offs[g][p], OWN)
                            rdma.start()
                            ag_pend[g][p] = rdma
                else:
                    k = t - 3
                    size = OWN << k
                    for p in range(3):
                        ag_pend[g][p].wait()
                        low = lows[ORDER[p][2 - k]]
                        offs[g][p] = jnp.where(low, offs[g][p],
                                               offs[g][p] - size)
                    if k < 2:
                        for p in range(3):
                            rdma = make_ag(g, k + 1, p, offs[g][p], size * 2)
                            rdma.start()
                            ag_pend[g][p] = rdma
                    else:
                        rb = g * GROWS
                        for p in range(3):
                            c0, nc = PART_COLS[p]
                            out_ref[rb:rb + GROWS, c0:c0 + nc] = (
                                g_refs[p][rb:rb + GROWS, :]
                                .astype(jnp.float32))

    flat = pl.pallas_call(
        body,
        out_shape=jax.ShapeDtypeStruct((ROWS, D), jnp.float32),
        in_specs=[pl.BlockSpec(memory_space=pltpu.VMEM)] * 5,
        out_specs=pl.BlockSpec(memory_space=pltpu.VMEM),
        scratch_shapes=[
            pltpu.VMEM((ROWS, D), jnp.float32),
            pltpu.VMEM((ROWS, D), jnp.float32),
            pltpu.VMEM((ROWS, HKV_LOC * DH), jnp.float32),
            pltpu.VMEM((ROWS, HKV_LOC * DH), jnp.float32),
            pltpu.VMEM((ROWS, 384), jnp.float32),
            pltpu.VMEM((ROWS, 384), jnp.float32),
            pltpu.VMEM((ROWS, 256), jnp.float32),
            pltpu.VMEM((G * GRECV, 384), jnp.bfloat16),
            pltpu.VMEM((G * GRECV, 384), jnp.bfloat16),
            pltpu.VMEM((G * GRECV, 256), jnp.bfloat16),
            pltpu.VMEM((ROWS, 384), jnp.bfloat16),
            pltpu.VMEM((ROWS, 384), jnp.bfloat16),
            pltpu.VMEM((ROWS, 256), jnp.bfloat16),
            pltpu.VMEM((G * GRECV, 384), jnp.bfloat16),
            pltpu.VMEM((G * GRECV, 384), jnp.bfloat16),
            pltpu.VMEM((G * GRECV, 256), jnp.bfloat16),
            pltpu.SemaphoreType.DMA((G * 18,)),
            pltpu.SemaphoreType.DMA((G * 18,)),
        ],
        compiler_params=pltpu.CompilerParams(collective_id=0),
    )(x, Wq, Wo, wk_s, wv_s)
    return flat.reshape(B, SQ, D)


# device time: 40992 ns/iter; 1.0357x vs baseline; 1.0242x over previous
import jax
import jax.numpy as jnp
from jax import lax
from jax.experimental import pallas as pl
from jax.experimental.pallas import tpu as pltpu

N_DEV = 8
B, SQ, D = 4, 256, 1024
ROWS = B * SQ
HQ_LOC, HKV_LOC, DH = 8, 2, 128
SCALE = 0.08838834764831843

G = 4
GROWS = ROWS // G
OWN = GROWS // N_DEV

PART_COLS = ((0, 384), (384, 384), (768, 256))
ORDER = ((0, 1, 2), (1, 2, 0), (2, 0, 1))
RS_SLOT = (0, 128, 192)
GRECV = 224


def kernel(x, Wq, Wo, Wk, Wv):
    idx = lax.axis_index("i")
    wk_s = lax.dynamic_slice_in_dim(Wk, idx * (HKV_LOC * DH), HKV_LOC * DH, axis=1)
    wv_s = lax.dynamic_slice_in_dim(Wv, idx * (HKV_LOC * DH), HKV_LOC * DH, axis=1)

    def body(x_ref, wq_ref, wo_ref, wk_ref, wv_ref, out_ref,
             attn_ref, q_ref, k_ref, v_ref,
             p0, p1, p2, r0, r1, r2, g0, g1, g2,
             send_sems, recv_sems):
        p_refs = (p0, p1, p2)
        r_refs = (r0, r1, r2)
        g_refs = (g0, g1, g2)

        i = lax.axis_index("i")
        partners = (
            jnp.bitwise_xor(i, 1),
            jnp.bitwise_xor(i, 3),
            jnp.bitwise_xor(i, 4),
        )
        lows = (
            jnp.bitwise_and(jnp.bitwise_xor(i, i >> 1), 1) == 0,
            jnp.bitwise_and(i >> 1, 1) == 0,
            jnp.bitwise_and(i >> 2, 1) == 0,
        )

        barrier_sem = pltpu.get_barrier_semaphore()
        for pd in partners:
            pl.semaphore_signal(barrier_sem, inc=1, device_id=(pd,),
                                device_id_type=pl.DeviceIdType.MESH)
        pl.semaphore_wait(barrier_sem, 3)

        def rs_sem(g, k, p):
            return g * 18 + k * 3 + p

        def ag_sem(g, k, p):
            return g * 18 + 9 + k * 3 + p

        def make_rs(g, k, p, src_off, half):
            return pltpu.make_async_remote_copy(
                src_ref=p_refs[p].at[pl.ds(pl.multiple_of(src_off, OWN), half), :],
                dst_ref=r_refs[p].at[pl.ds(g * GRECV + RS_SLOT[k], half), :],
                send_sem=send_sems.at[rs_sem(g, k, p)],
                recv_sem=recv_sems.at[rs_sem(g, k, p)],
                device_id=(partners[ORDER[p][k]],),
                device_id_type=pl.DeviceIdType.MESH,
            )

        def make_ag(g, k, p, off, size):
            return pltpu.make_async_remote_copy(
                src_ref=g_refs[p].at[pl.ds(pl.multiple_of(off, OWN), size), :],
                dst_ref=g_refs[p].at[pl.ds(pl.multiple_of(off, OWN), size), :],
                send_sem=send_sems.at[ag_sem(g, k, p)],
                recv_sem=recv_sems.at[ag_sem(g, k, p)],
                device_id=(partners[ORDER[p][2 - k]],),
                device_id_type=pl.DeviceIdType.MESH,
            )

        offs = [[jnp.int32(g * GROWS)] * 3 for g in range(G)]
        rs_pend = [[None] * 3 for _ in range(G)]
        ag_pend = [[None] * 3 for _ in range(G)]

        xf = x_ref[...].reshape(ROWS, D)
        q_ref[:] = jnp.dot(xf, wq_ref[:], preferred_element_type=jnp.float32)
        k_ref[:] = jnp.dot(xf, wk_ref[:], preferred_element_type=jnp.float32)
        v_ref[:] = jnp.dot(xf, wv_ref[:], preferred_element_type=jnp.float32)

        for g in range(G):
            b = g
            qb = q_ref[b * SQ:(b + 1) * SQ, :]
            kb = k_ref[b * SQ:(b + 1) * SQ, :]
            vb = v_ref[b * SQ:(b + 1) * SQ, :]
            for h in range(HQ_LOC):
                kv = h // 4
                q = qb[:, h * DH:(h + 1) * DH]
                k_ = kb[:, kv * DH:(kv + 1) * DH]
                v_ = vb[:, kv * DH:(kv + 1) * DH]
                s = lax.dot_general(
                    q, k_, (((1,), (1,)), ((), ())),
                    preferred_element_type=jnp.float32,
                ) * SCALE
                m = jnp.max(s, axis=1, keepdims=True)
                pe = jnp.exp(s - m)
                l = jnp.sum(pe, axis=1, keepdims=True)
                o = jnp.dot(pe, v_, preferred_element_type=jnp.float32) / l
                attn_ref[b * SQ:(b + 1) * SQ, h * DH:(h + 1) * DH] = o
            rb = g * GROWS
            half = GROWS // 2
            for p in range(3):
                c0, nc = PART_COLS[p]
                p_refs[p][rb:rb + GROWS, :] = jnp.dot(
                    attn_ref[rb:rb + GROWS, :], wo_ref[:, c0:c0 + nc],
                    preferred_element_type=jnp.float32
                ).astype(jnp.bfloat16)
                low = lows[ORDER[p][0]]
                send_off = jnp.where(low, rb + half, rb)
                rdma = make_rs(g, 0, p, send_off, half)
                rdma.start()
                rs_pend[g][p] = (rdma, jnp.where(low, rb, rb + half))

        for t in range(6):
            for g in range(G):
                if t < 3:
                    k = t
                    half = GROWS >> (k + 1)
                    for p in range(3):
                        rdma, keep_off = rs_pend[g][p]
                        rdma.wait()
                        p_refs[p][pl.ds(pl.multiple_of(keep_off, OWN), half), :] = (
                            p_refs[p][pl.ds(pl.multiple_of(keep_off, OWN), half), :]
                            + r_refs[p][pl.ds(g * GRECV + RS_SLOT[k], half), :]
                        )
                        offs[g][p] = keep_off
                    if k < 2:
                        nhalf = half // 2
                        for p in range(3):
                            low = lows[ORDER[p][k + 1]]
                            send_off = jnp.where(low, offs[g][p] + nhalf,
                                                 offs[g][p])
                            rdma = make_rs(g, k + 1, p, send_off, nhalf)
                            rdma.start()
                            rs_pend[g][p] = (rdma,
                                             jnp.where(low, offs[g][p],
                                                       offs[g][p] + nhalf))
                    else:
                        for p in range(3):
                            g_refs[p][pl.ds(pl.multiple_of(offs[g][p], OWN), OWN), :] = (
                                p_refs[p][pl.ds(pl.multiple_of(offs[g][p], OWN), OWN), :])
                            rdma = make_ag(g, 0, p, offs[g][p], OWN)
                            rdma.start()
                            ag_pend[g][p] = rdma
                else:
                    k = t - 3
                    size = OWN << k
                    for p in range(3):
                        ag_pend[g][p].wait()
                        low = lows[ORDER[p][2 - k]]
                        offs[g][p] = jnp.where(low, offs[g][p],
                                               offs[g][p] - size)
                    if k < 2:
                        for p in range(3):
                            rdma = make_ag(g, k + 1, p, offs[g][p], size * 2)
                            rdma.start()
                            ag_pend[g][p] = rdma
                    else:
                        rb = g * GROWS
                        for p in range(3):
                            c0, nc = PART_COLS[p]
                            out_ref[rb:rb + GROWS, c0:c0 + nc] = (
                                g_refs[p][rb:rb + GROWS, :]
                                .astype(jnp.float32))

    flat = pl.pallas_call(
        body,
        out_shape=jax.ShapeDtypeStruct((ROWS, D), jnp.float32),
        in_specs=[pl.BlockSpec(memory_space=pltpu.VMEM)] * 5,
        out_specs=pl.BlockSpec(memory_space=pltpu.VMEM),
        scratch_shapes=[
            pltpu.VMEM((ROWS, D), jnp.float32),
            pltpu.VMEM((ROWS, D), jnp.float32),
            pltpu.VMEM((ROWS, HKV_LOC * DH), jnp.float32),
            pltpu.VMEM((ROWS, HKV_LOC * DH), jnp.float32),
            pltpu.VMEM((ROWS, 384), jnp.bfloat16),
            pltpu.VMEM((ROWS, 384), jnp.bfloat16),
            pltpu.VMEM((ROWS, 256), jnp.bfloat16),
            pltpu.VMEM((G * GRECV, 384), jnp.bfloat16),
            pltpu.VMEM((G * GRECV, 384), jnp.bfloat16),
            pltpu.VMEM((G * GRECV, 256), jnp.bfloat16),
            pltpu.VMEM((ROWS, 384), jnp.bfloat16),
            pltpu.VMEM((ROWS, 384), jnp.bfloat16),
            pltpu.VMEM((ROWS, 256), jnp.bfloat16),
            pltpu.SemaphoreType.DMA((G * 18,)),
            pltpu.SemaphoreType.DMA((G * 18,)),
        ],
        compiler_params=pltpu.CompilerParams(collective_id=0),
    )(x, Wq, Wo, wk_s, wv_s)
    return flat.reshape(B, SQ, D)
